# baseline (speedup 1.0000x reference)
import functools

import jax
import jax.numpy as jnp
from jax import lax
from jax.experimental import pallas as pl
from jax.experimental.pallas import tpu as pltpu
from jax.experimental.pallas import tpu_sc as plsc

_N = 10000
_D = 128
_E = 320000
_NC = 2
_NS = 16
_NW = _NC * _NS
_K = 128
_EPW = _E // _NW
_NCH = _EPW // _K
_TAIL = _EPW - _NCH * _K
_NPAD = 10240
_RPZ = _NPAD // _NS


def _make_segsum():
    mesh = plsc.VectorSubcoreMesh(core_axis_name="c", subcore_axis_name="s")

    @functools.partial(
        pl.kernel,
        mesh=mesh,
        out_type=jax.ShapeDtypeStruct((_NC, _NPAD, _D), jnp.float32),
        scratch_types=[
            pltpu.VMEM((4, _K), jnp.int32),
            pltpu.VMEM((4, _K), jnp.int32),
            pltpu.VMEM((_TAIL,), jnp.int32),
            pltpu.VMEM((_TAIL,), jnp.int32),
            pltpu.VMEM((_K, _D), jnp.float32),
            pltpu.VMEM((_K, _D), jnp.float32),
            pltpu.VMEM((_TAIL, _D), jnp.float32),
            pltpu.VMEM_SHARED((_NPAD, _D), jnp.float32),
            pltpu.SemaphoreType.DMA,
            pltpu.SemaphoreType.DMA,
            pltpu.SemaphoreType.DMA,
        ],
    )
    def seg(h_hbm, src_hbm, dst_hbm, z_hbm, out_hbm,
            sidx, didx, sidx_t, didx_t, rows0, rows1, rows_t, agg_sh,
            gs0, gs1, isem):
        c = lax.axis_index("c")
        s = lax.axis_index("s")
        gwid = s * _NC + c
        ebase = gwid * _EPW
        pltpu.sync_copy(z_hbm, agg_sh.at[pl.ds(s * _RPZ, _RPZ)])
        plsc.subcore_barrier()

        rows = (rows0, rows1)
        gs = (gs0, gs1)

        def drain(buf, sem):
            pltpu.make_async_copy(h_hbm.at[pl.ds(0, buf.shape[0])],
                                  buf, sem).wait()

        for b in range(2):
            pltpu.sync_copy(src_hbm.at[pl.ds(ebase + b * _K, _K)], sidx.at[b])
            pltpu.sync_copy(dst_hbm.at[pl.ds(ebase + b * _K, _K)], didx.at[b])
            pltpu.async_copy(h_hbm.at[sidx.at[b]], rows[b], gs[b])

        def body(jj, carry):
            for b in range(2):
                j = jj * 2 + b
                s_cur = lax.rem(j, 4)
                s_pre = lax.rem(j + 2, 4)
                drain(rows[b], gs[b])
                @pl.when(j + 2 < _NCH)
                def _():
                    pltpu.async_copy(
                        src_hbm.at[pl.ds(ebase + (j + 2) * _K, _K)],
                        sidx.at[s_pre], isem)
                    pltpu.async_copy(
                        dst_hbm.at[pl.ds(ebase + (j + 2) * _K, _K)],
                        didx.at[s_pre], isem)
                pltpu.sync_copy(rows[b], agg_sh.at[didx.at[s_cur]], add=True)
                @pl.when(j + 2 < _NCH)
                def _():
                    pltpu.make_async_copy(
                        src_hbm.at[pl.ds(0, _K)], sidx.at[s_pre], isem).wait()
                    pltpu.make_async_copy(
                        src_hbm.at[pl.ds(0, _K)], didx.at[s_pre], isem).wait()
                    pltpu.async_copy(
                        h_hbm.at[sidx.at[s_pre]], rows[b], gs[b])
            return carry

        lax.fori_loop(0, _NCH // 2, body, 0)
        tbase = ebase + _NCH * _K
        pltpu.sync_copy(src_hbm.at[pl.ds(tbase, _TAIL)], sidx_t)
        pltpu.sync_copy(dst_hbm.at[pl.ds(tbase, _TAIL)], didx_t)
        pltpu.async_copy(h_hbm.at[sidx_t], rows_t, gs0).wait()
        pltpu.sync_copy(rows_t, agg_sh.at[didx_t], add=True)
        plsc.subcore_barrier()
        pltpu.sync_copy(agg_sh.at[pl.ds(s * _RPZ, _RPZ)],
                        out_hbm.at[c, pl.ds(s * _RPZ, _RPZ)])

    return seg


_segsum = _make_segsum()


_R = 10000


def _lin_body(x_ref, w_ref, b_ref, o_ref):
    o_ref[...] = (
        jnp.dot(x_ref[...], w_ref[...], preferred_element_type=jnp.float32)
        + b_ref[...]
    )


def _linear(x, w, b2d):
    n, din = x.shape
    dout = w.shape[1]
    return pl.pallas_call(
        _lin_body,
        grid=(n // _R,),
        in_specs=[
            pl.BlockSpec((_R, din), lambda i: (i, 0)),
            pl.BlockSpec((din, dout), lambda i: (0, 0)),
            pl.BlockSpec((1, dout), lambda i: (0, 0)),
        ],
        out_specs=pl.BlockSpec((_R, dout), lambda i: (i, 0)),
        out_shape=jax.ShapeDtypeStruct((n, dout), jnp.float32),
    )(x, w, b2d)


def _mlp_body(h_ref, p0_ref, p1_ref, e_ref, w1_ref, b1_ref, w2_ref, b2_ref,
              o_ref):
    s = h_ref[...] * e_ref[...] + p0_ref[0] + p1_ref[0]
    t = jnp.maximum(
        jnp.dot(s, w1_ref[...], preferred_element_type=jnp.float32)
        + b1_ref[...], 0.0)
    o_ref[...] = (
        jnp.dot(t, w2_ref[...], preferred_element_type=jnp.float32)
        + b2_ref[...]
    )


def _gin_mlp(h, parts, e_row, w1, b1_2d, w2, b2_2d):
    row = lambda i: (i, 0)
    fixed = lambda i: (0, 0)
    part0 = lambda i: (0, i, 0)
    part1 = lambda i: (1, i, 0)
    return pl.pallas_call(
        _mlp_body,
        grid=(_N // _R,),
        in_specs=[
            pl.BlockSpec((_R, _D), row),
            pl.BlockSpec((1, _R, _D), part0),
            pl.BlockSpec((1, _R, _D), part1),
            pl.BlockSpec((1, _D), fixed),
            pl.BlockSpec((_D, _D), fixed),
            pl.BlockSpec((1, _D), fixed),
            pl.BlockSpec((_D, _D), fixed),
            pl.BlockSpec((1, _D), fixed),
        ],
        out_specs=pl.BlockSpec((_R, _D), row),
        out_shape=jax.ShapeDtypeStruct((_N, _D), jnp.float32),
    )(h, parts, parts, e_row, w1, b1_2d, w2, b2_2d)


def _final_body(h0_ref, h1_ref, q0_ref, q1_ref, e_ref, w1_ref, b1_ref,
                w2_ref, b2_ref, wo_ref, bo_ref, o_ref):
    s = h1_ref[...] * e_ref[...] + q0_ref[0] + q1_ref[0]
    t = jnp.maximum(
        jnp.dot(s, w1_ref[...], preferred_element_type=jnp.float32)
        + b1_ref[...], 0.0)
    h2 = (jnp.dot(t, w2_ref[...], preferred_element_type=jnp.float32)
          + b2_ref[...])
    logits = (
        jnp.dot(h0_ref[...], wo_ref[0:_D, :],
                preferred_element_type=jnp.float32)
        + jnp.dot(h1_ref[...], wo_ref[_D:2 * _D, :],
                  preferred_element_type=jnp.float32)
        + jnp.dot(h2, wo_ref[2 * _D:3 * _D, :],
                  preferred_element_type=jnp.float32)
        + bo_ref[...]
    )
    m = jnp.max(logits, axis=-1, keepdims=True)
    ex = jnp.exp(logits - m)
    lse = jnp.log(jnp.sum(ex, axis=-1, keepdims=True)) + m
    o_ref[...] = logits - lse


def _final(h0, h1, parts, e_row, w1, b1_2d, w2, b2_2d, wo, bo_2d):
    row = lambda i: (i, 0)
    fixed = lambda i: (0, 0)
    part0 = lambda i: (0, i, 0)
    part1 = lambda i: (1, i, 0)
    return pl.pallas_call(
        _final_body,
        grid=(_N // _R,),
        in_specs=[
            pl.BlockSpec((_R, _D), row),
            pl.BlockSpec((_R, _D), row),
            pl.BlockSpec((1, _R, _D), part0),
            pl.BlockSpec((1, _R, _D), part1),
            pl.BlockSpec((1, _D), fixed),
            pl.BlockSpec((_D, _D), fixed),
            pl.BlockSpec((1, _D), fixed),
            pl.BlockSpec((_D, _D), fixed),
            pl.BlockSpec((1, _D), fixed),
            pl.BlockSpec((3 * _D, _D), fixed),
            pl.BlockSpec((1, _D), fixed),
        ],
        out_specs=pl.BlockSpec((_R, _D), row),
        out_shape=jax.ShapeDtypeStruct((_N, _D), jnp.float32),
    )(h0, h1, parts, parts, e_row, w1, b1_2d, w2, b2_2d, wo, bo_2d)


def kernel(X, A, in_W, in_b, eps0, W1_0, b1_0, W2_0, b2_0,
           eps1, W1_1, b1_1, W2_1, b2_1, out_W, out_b):
    src = A[0]
    dst = A[1]
    zeros = jnp.zeros((_RPZ, _D), jnp.float32)
    e0 = jnp.full((1, _D), 1.0 + eps0, jnp.float32)
    e1 = jnp.full((1, _D), 1.0 + eps1, jnp.float32)

    h0 = _linear(X, in_W, in_b.reshape(1, -1))
    parts0 = _segsum(h0, src, dst, zeros)
    h1 = _gin_mlp(h0, parts0, e0,
                  W1_0, b1_0.reshape(1, -1), W2_0, b2_0.reshape(1, -1))
    parts1 = _segsum(h1, src, dst, zeros)
    return _final(h0, h1, parts1, e1,
                  W1_1, b1_1.reshape(1, -1), W2_1, b2_1.reshape(1, -1),
                  out_W, out_b.reshape(1, -1))


# speedup vs baseline: 1.0266x; 1.0266x over previous
import functools

import jax
import jax.numpy as jnp
from jax import lax
from jax.experimental import pallas as pl
from jax.experimental.pallas import tpu as pltpu
from jax.experimental.pallas import tpu_sc as plsc

_N = 10000
_D = 128
_E = 320000
_NC = 2
_NS = 16
_NW = _NC * _NS
_K = 128
_EPW = _E // _NW
_NCH = _EPW // _K
_TAIL = _EPW - _NCH * _K
_NPAD = 10240
_RPZ = _NPAD // _NS


def _make_segsum():
    mesh = plsc.VectorSubcoreMesh(core_axis_name="c", subcore_axis_name="s")

    @functools.partial(
        pl.kernel,
        mesh=mesh,
        out_type=jax.ShapeDtypeStruct((_NC, _NPAD, _D), jnp.float32),
        scratch_types=[
            pltpu.VMEM((4, _K), jnp.int32),
            pltpu.VMEM((4, _K), jnp.int32),
            pltpu.VMEM((_TAIL,), jnp.int32),
            pltpu.VMEM((_TAIL,), jnp.int32),
            pltpu.VMEM((_K, _D), jnp.float32),
            pltpu.VMEM((_K, _D), jnp.float32),
            pltpu.VMEM((_TAIL, _D), jnp.float32),
            pltpu.VMEM_SHARED((_NPAD, _D), jnp.float32),
            pltpu.SemaphoreType.DMA,
            pltpu.SemaphoreType.DMA,
            pltpu.SemaphoreType.DMA,
        ],
    )
    def seg(h_hbm, src_hbm, dst_hbm, z_hbm, out_hbm,
            sidx, didx, sidx_t, didx_t, rows0, rows1, rows_t, agg_sh,
            gs0, gs1, isem):
        c = lax.axis_index("c")
        s = lax.axis_index("s")
        gwid = s * _NC + c
        ebase = gwid * _EPW
        rows = (rows0, rows1)
        gs = (gs0, gs1)

        def drain(buf, sem):
            pltpu.make_async_copy(h_hbm.at[pl.ds(0, buf.shape[0])],
                                  buf, sem).wait()

        for b in range(2):
            pltpu.sync_copy(src_hbm.at[pl.ds(ebase + b * _K, _K)], sidx.at[b])
            pltpu.sync_copy(dst_hbm.at[pl.ds(ebase + b * _K, _K)], didx.at[b])
            pltpu.async_copy(h_hbm.at[sidx.at[b]], rows[b], gs[b])

        pltpu.sync_copy(z_hbm, agg_sh.at[pl.ds(s * _RPZ, _RPZ)])
        plsc.subcore_barrier()

        def body(jj, carry):
            for b in range(2):
                j = jj * 2 + b
                s_cur = lax.rem(j, 4)
                s_pre = lax.rem(j + 2, 4)
                @pl.when(j + 2 < _NCH)
                def _():
                    pltpu.async_copy(
                        src_hbm.at[pl.ds(ebase + (j + 2) * _K, _K)],
                        sidx.at[s_pre], isem)
                    pltpu.async_copy(
                        dst_hbm.at[pl.ds(ebase + (j + 2) * _K, _K)],
                        didx.at[s_pre], isem)
                drain(rows[b], gs[b])
                pltpu.sync_copy(rows[b], agg_sh.at[didx.at[s_cur]], add=True)
                @pl.when(j + 2 < _NCH)
                def _():
                    pltpu.make_async_copy(
                        src_hbm.at[pl.ds(0, _K)], sidx.at[s_pre], isem).wait()
                    pltpu.make_async_copy(
                        src_hbm.at[pl.ds(0, _K)], didx.at[s_pre], isem).wait()
                    pltpu.async_copy(
                        h_hbm.at[sidx.at[s_pre]], rows[b], gs[b])
            return carry

        lax.fori_loop(0, _NCH // 2, body, 0)
        tbase = ebase + _NCH * _K
        pltpu.sync_copy(src_hbm.at[pl.ds(tbase, _TAIL)], sidx_t)
        pltpu.sync_copy(dst_hbm.at[pl.ds(tbase, _TAIL)], didx_t)
        pltpu.async_copy(h_hbm.at[sidx_t], rows_t, gs0).wait()
        pltpu.sync_copy(rows_t, agg_sh.at[didx_t], add=True)
        plsc.subcore_barrier()
        pltpu.sync_copy(agg_sh.at[pl.ds(s * _RPZ, _RPZ)],
                        out_hbm.at[c, pl.ds(s * _RPZ, _RPZ)])

    return seg


_segsum = _make_segsum()


_R = 5000


def _lin_body(x_ref, w_ref, b_ref, o_ref):
    o_ref[...] = (
        jnp.dot(x_ref[...], w_ref[...], preferred_element_type=jnp.float32)
        + b_ref[...]
    )


def _linear(x, w, b2d):
    n, din = x.shape
    dout = w.shape[1]
    return pl.pallas_call(
        _lin_body,
        grid=(n // _R,),
        in_specs=[
            pl.BlockSpec((_R, din), lambda i: (i, 0)),
            pl.BlockSpec((din, dout), lambda i: (0, 0)),
            pl.BlockSpec((1, dout), lambda i: (0, 0)),
        ],
        out_specs=pl.BlockSpec((_R, dout), lambda i: (i, 0)),
        out_shape=jax.ShapeDtypeStruct((n, dout), jnp.float32),
    )(x, w, b2d)


def _mlp_body(h_ref, p0_ref, p1_ref, e_ref, w1_ref, b1_ref, w2_ref, b2_ref,
              o_ref):
    s = h_ref[...] * e_ref[...] + p0_ref[0] + p1_ref[0]
    t = jnp.maximum(
        jnp.dot(s, w1_ref[...], preferred_element_type=jnp.float32)
        + b1_ref[...], 0.0)
    o_ref[...] = (
        jnp.dot(t, w2_ref[...], preferred_element_type=jnp.float32)
        + b2_ref[...]
    )


def _gin_mlp(h, parts, e_row, w1, b1_2d, w2, b2_2d):
    row = lambda i: (i, 0)
    fixed = lambda i: (0, 0)
    part0 = lambda i: (0, i, 0)
    part1 = lambda i: (1, i, 0)
    return pl.pallas_call(
        _mlp_body,
        grid=(_N // _R,),
        in_specs=[
            pl.BlockSpec((_R, _D), row),
            pl.BlockSpec((1, _R, _D), part0),
            pl.BlockSpec((1, _R, _D), part1),
            pl.BlockSpec((1, _D), fixed),
            pl.BlockSpec((_D, _D), fixed),
            pl.BlockSpec((1, _D), fixed),
            pl.BlockSpec((_D, _D), fixed),
            pl.BlockSpec((1, _D), fixed),
        ],
        out_specs=pl.BlockSpec((_R, _D), row),
        out_shape=jax.ShapeDtypeStruct((_N, _D), jnp.float32),
    )(h, parts, parts, e_row, w1, b1_2d, w2, b2_2d)


def _final_body(h0_ref, h1_ref, q0_ref, q1_ref, e_ref, w1_ref, b1_ref,
                w2_ref, b2_ref, wo_ref, bo_ref, o_ref):
    s = h1_ref[...] * e_ref[...] + q0_ref[0] + q1_ref[0]
    t = jnp.maximum(
        jnp.dot(s, w1_ref[...], preferred_element_type=jnp.float32)
        + b1_ref[...], 0.0)
    h2 = (jnp.dot(t, w2_ref[...], preferred_element_type=jnp.float32)
          + b2_ref[...])
    logits = (
        jnp.dot(h0_ref[...], wo_ref[0:_D, :],
                preferred_element_type=jnp.float32)
        + jnp.dot(h1_ref[...], wo_ref[_D:2 * _D, :],
                  preferred_element_type=jnp.float32)
        + jnp.dot(h2, wo_ref[2 * _D:3 * _D, :],
                  preferred_element_type=jnp.float32)
        + bo_ref[...]
    )
    m = jnp.max(logits, axis=-1, keepdims=True)
    ex = jnp.exp(logits - m)
    lse = jnp.log(jnp.sum(ex, axis=-1, keepdims=True)) + m
    o_ref[...] = logits - lse


def _final(h0, h1, parts, e_row, w1, b1_2d, w2, b2_2d, wo, bo_2d):
    row = lambda i: (i, 0)
    fixed = lambda i: (0, 0)
    part0 = lambda i: (0, i, 0)
    part1 = lambda i: (1, i, 0)
    return pl.pallas_call(
        _final_body,
        grid=(_N // _R,),
        in_specs=[
            pl.BlockSpec((_R, _D), row),
            pl.BlockSpec((_R, _D), row),
            pl.BlockSpec((1, _R, _D), part0),
            pl.BlockSpec((1, _R, _D), part1),
            pl.BlockSpec((1, _D), fixed),
            pl.BlockSpec((_D, _D), fixed),
            pl.BlockSpec((1, _D), fixed),
            pl.BlockSpec((_D, _D), fixed),
            pl.BlockSpec((1, _D), fixed),
            pl.BlockSpec((3 * _D, _D), fixed),
            pl.BlockSpec((1, _D), fixed),
        ],
        out_specs=pl.BlockSpec((_R, _D), row),
        out_shape=jax.ShapeDtypeStruct((_N, _D), jnp.float32),
    )(h0, h1, parts, parts, e_row, w1, b1_2d, w2, b2_2d, wo, bo_2d)


def kernel(X, A, in_W, in_b, eps0, W1_0, b1_0, W2_0, b2_0,
           eps1, W1_1, b1_1, W2_1, b2_1, out_W, out_b):
    src = A[0]
    dst = A[1]
    zeros = jnp.zeros((_RPZ, _D), jnp.float32)
    e0 = jnp.full((1, _D), 1.0 + eps0, jnp.float32)
    e1 = jnp.full((1, _D), 1.0 + eps1, jnp.float32)

    h0 = _linear(X, in_W, in_b.reshape(1, -1))
    parts0 = _segsum(h0, src, dst, zeros)
    h1 = _gin_mlp(h0, parts0, e0,
                  W1_0, b1_0.reshape(1, -1), W2_0, b2_0.reshape(1, -1))
    parts1 = _segsum(h1, src, dst, zeros)
    return _final(h0, h1, parts1, e1,
                  W1_1, b1_1.reshape(1, -1), W2_1, b2_1.reshape(1, -1),
                  out_W, out_b.reshape(1, -1))


# speedup vs baseline: 1.0307x; 1.0039x over previous
import functools

import jax
import jax.numpy as jnp
from jax import lax
from jax.experimental import pallas as pl
from jax.experimental.pallas import tpu as pltpu
from jax.experimental.pallas import tpu_sc as plsc

_N = 10000
_D = 128
_E = 320000
_NC = 2
_NS = 16
_NW = _NC * _NS
_K = 128
_EPW = _E // _NW
_NCH = _EPW // _K
_TAIL = _EPW - _NCH * _K
_NPAD = 10240
_RPZ = _NPAD // _NS


def _make_segsum():
    mesh = plsc.VectorSubcoreMesh(core_axis_name="c", subcore_axis_name="s")

    @functools.partial(
        pl.kernel,
        mesh=mesh,
        out_type=jax.ShapeDtypeStruct((_NC, _NPAD, _D), jnp.float32),
        scratch_types=[
            pltpu.VMEM((4, _K), jnp.int32),
            pltpu.VMEM((4, _K), jnp.int32),
            pltpu.VMEM((_TAIL,), jnp.int32),
            pltpu.VMEM((_TAIL,), jnp.int32),
            pltpu.VMEM((_K, _D), jnp.float32),
            pltpu.VMEM((_K, _D), jnp.float32),
            pltpu.VMEM((_TAIL, _D), jnp.float32),
            pltpu.VMEM_SHARED((_NPAD, _D), jnp.float32),
            pltpu.SemaphoreType.DMA,
            pltpu.SemaphoreType.DMA,
            pltpu.SemaphoreType.DMA,
        ],
    )
    def seg(h_hbm, src_hbm, dst_hbm, z_hbm, out_hbm,
            sidx, didx, sidx_t, didx_t, rows0, rows1, rows_t, agg_sh,
            gs0, gs1, isem):
        c = lax.axis_index("c")
        s = lax.axis_index("s")
        gwid = s * _NC + c
        ebase = gwid * _EPW
        rows = (rows0, rows1)
        gs = (gs0, gs1)

        def drain(buf, sem):
            pltpu.make_async_copy(h_hbm.at[pl.ds(0, buf.shape[0])],
                                  buf, sem).wait()

        for b in range(2):
            pltpu.sync_copy(src_hbm.at[pl.ds(ebase + b * _K, _K)], sidx.at[b])
            pltpu.sync_copy(dst_hbm.at[pl.ds(ebase + b * _K, _K)], didx.at[b])
            pltpu.async_copy(h_hbm.at[sidx.at[b]], rows[b], gs[b])
        tbase = ebase + _NCH * _K
        pltpu.sync_copy(src_hbm.at[pl.ds(tbase, _TAIL)], sidx_t)
        pltpu.sync_copy(dst_hbm.at[pl.ds(tbase, _TAIL)], didx_t)
        pltpu.async_copy(h_hbm.at[sidx_t], rows_t, isem)

        pltpu.sync_copy(z_hbm, agg_sh.at[pl.ds(s * _RPZ, _RPZ)])
        plsc.subcore_barrier()
        pltpu.make_async_copy(h_hbm.at[pl.ds(0, _TAIL)], rows_t, isem).wait()
        pltpu.sync_copy(rows_t, agg_sh.at[didx_t], add=True)

        def body(jj, carry):
            for b in range(2):
                j = jj * 2 + b
                s_cur = lax.rem(j, 4)
                s_pre = lax.rem(j + 2, 4)
                @pl.when(j + 2 < _NCH)
                def _():
                    pltpu.async_copy(
                        src_hbm.at[pl.ds(ebase + (j + 2) * _K, _K)],
                        sidx.at[s_pre], isem)
                    pltpu.async_copy(
                        dst_hbm.at[pl.ds(ebase + (j + 2) * _K, _K)],
                        didx.at[s_pre], isem)
                drain(rows[b], gs[b])
                pltpu.sync_copy(rows[b], agg_sh.at[didx.at[s_cur]], add=True)
                @pl.when(j + 2 < _NCH)
                def _():
                    pltpu.make_async_copy(
                        src_hbm.at[pl.ds(0, _K)], sidx.at[s_pre], isem).wait()
                    pltpu.make_async_copy(
                        src_hbm.at[pl.ds(0, _K)], didx.at[s_pre], isem).wait()
                    pltpu.async_copy(
                        h_hbm.at[sidx.at[s_pre]], rows[b], gs[b])
            return carry

        lax.fori_loop(0, _NCH // 2, body, 0)
        plsc.subcore_barrier()
        pltpu.sync_copy(agg_sh.at[pl.ds(s * _RPZ, _RPZ)],
                        out_hbm.at[c, pl.ds(s * _RPZ, _RPZ)])

    return seg


_segsum = _make_segsum()


_R = 5000


def _lin_body(x_ref, w_ref, b_ref, o_ref):
    o_ref[...] = (
        jnp.dot(x_ref[...], w_ref[...], preferred_element_type=jnp.float32)
        + b_ref[...]
    )


def _linear(x, w, b2d):
    n, din = x.shape
    dout = w.shape[1]
    return pl.pallas_call(
        _lin_body,
        grid=(n // _R,),
        in_specs=[
            pl.BlockSpec((_R, din), lambda i: (i, 0)),
            pl.BlockSpec((din, dout), lambda i: (0, 0)),
            pl.BlockSpec((1, dout), lambda i: (0, 0)),
        ],
        out_specs=pl.BlockSpec((_R, dout), lambda i: (i, 0)),
        out_shape=jax.ShapeDtypeStruct((n, dout), jnp.float32),
    )(x, w, b2d)


def _mlp_body(h_ref, p0_ref, p1_ref, e_ref, w1_ref, b1_ref, w2_ref, b2_ref,
              o_ref):
    s = h_ref[...] * e_ref[...] + p0_ref[0] + p1_ref[0]
    t = jnp.maximum(
        jnp.dot(s, w1_ref[...], preferred_element_type=jnp.float32)
        + b1_ref[...], 0.0)
    o_ref[...] = (
        jnp.dot(t, w2_ref[...], preferred_element_type=jnp.float32)
        + b2_ref[...]
    )


def _gin_mlp(h, parts, e_row, w1, b1_2d, w2, b2_2d):
    row = lambda i: (i, 0)
    fixed = lambda i: (0, 0)
    part0 = lambda i: (0, i, 0)
    part1 = lambda i: (1, i, 0)
    return pl.pallas_call(
        _mlp_body,
        grid=(_N // _R,),
        in_specs=[
            pl.BlockSpec((_R, _D), row),
            pl.BlockSpec((1, _R, _D), part0),
            pl.BlockSpec((1, _R, _D), part1),
            pl.BlockSpec((1, _D), fixed),
            pl.BlockSpec((_D, _D), fixed),
            pl.BlockSpec((1, _D), fixed),
            pl.BlockSpec((_D, _D), fixed),
            pl.BlockSpec((1, _D), fixed),
        ],
        out_specs=pl.BlockSpec((_R, _D), row),
        out_shape=jax.ShapeDtypeStruct((_N, _D), jnp.float32),
    )(h, parts, parts, e_row, w1, b1_2d, w2, b2_2d)


def _final_body(h0_ref, h1_ref, q0_ref, q1_ref, e_ref, w1_ref, b1_ref,
                w2_ref, b2_ref, wo_ref, bo_ref, o_ref):
    s = h1_ref[...] * e_ref[...] + q0_ref[0] + q1_ref[0]
    t = jnp.maximum(
        jnp.dot(s, w1_ref[...], preferred_element_type=jnp.float32)
        + b1_ref[...], 0.0)
    h2 = (jnp.dot(t, w2_ref[...], preferred_element_type=jnp.float32)
          + b2_ref[...])
    logits = (
        jnp.dot(h0_ref[...], wo_ref[0:_D, :],
                preferred_element_type=jnp.float32)
        + jnp.dot(h1_ref[...], wo_ref[_D:2 * _D, :],
                  preferred_element_type=jnp.float32)
        + jnp.dot(h2, wo_ref[2 * _D:3 * _D, :],
                  preferred_element_type=jnp.float32)
        + bo_ref[...]
    )
    m = jnp.max(logits, axis=-1, keepdims=True)
    ex = jnp.exp(logits - m)
    lse = jnp.log(jnp.sum(ex, axis=-1, keepdims=True)) + m
    o_ref[...] = logits - lse


def _final(h0, h1, parts, e_row, w1, b1_2d, w2, b2_2d, wo, bo_2d):
    row = lambda i: (i, 0)
    fixed = lambda i: (0, 0)
    part0 = lambda i: (0, i, 0)
    part1 = lambda i: (1, i, 0)
    return pl.pallas_call(
        _final_body,
        grid=(_N // _R,),
        in_specs=[
            pl.BlockSpec((_R, _D), row),
            pl.BlockSpec((_R, _D), row),
            pl.BlockSpec((1, _R, _D), part0),
            pl.BlockSpec((1, _R, _D), part1),
            pl.BlockSpec((1, _D), fixed),
            pl.BlockSpec((_D, _D), fixed),
            pl.BlockSpec((1, _D), fixed),
            pl.BlockSpec((_D, _D), fixed),
            pl.BlockSpec((1, _D), fixed),
            pl.BlockSpec((3 * _D, _D), fixed),
            pl.BlockSpec((1, _D), fixed),
        ],
        out_specs=pl.BlockSpec((_R, _D), row),
        out_shape=jax.ShapeDtypeStruct((_N, _D), jnp.float32),
    )(h0, h1, parts, parts, e_row, w1, b1_2d, w2, b2_2d, wo, bo_2d)


def kernel(X, A, in_W, in_b, eps0, W1_0, b1_0, W2_0, b2_0,
           eps1, W1_1, b1_1, W2_1, b2_1, out_W, out_b):
    src = A[0]
    dst = A[1]
    zeros = jnp.zeros((_RPZ, _D), jnp.float32)
    e0 = jnp.full((1, _D), 1.0 + eps0, jnp.float32)
    e1 = jnp.full((1, _D), 1.0 + eps1, jnp.float32)

    h0 = _linear(X, in_W, in_b.reshape(1, -1))
    parts0 = _segsum(h0, src, dst, zeros)
    h1 = _gin_mlp(h0, parts0, e0,
                  W1_0, b1_0.reshape(1, -1), W2_0, b2_0.reshape(1, -1))
    parts1 = _segsum(h1, src, dst, zeros)
    return _final(h0, h1, parts1, e1,
                  W1_1, b1_1.reshape(1, -1), W2_1, b2_1.reshape(1, -1),
                  out_W, out_b.reshape(1, -1))
